# baseline (speedup 1.0000x reference)
import functools

import jax
import jax.numpy as jnp
from jax import lax
from jax.experimental import pallas as pl
from jax.experimental.pallas import tpu as pltpu
from jax.experimental.pallas import tpu_sc as plsc

_B = 16384
_F = 26
_D = 16
_V = 1_000_000
_ROWS = _B * _F
_NC, _NS = 2, 16
_NW = _NC * _NS
_BC = _B // _NW
_GL = 128
_NGC = _BC // _GL
_TB = 2 * 8 * _BC
_FSTRIDE = _D * _B
_DTSTRIDE = 8 * _B

_TRK = 1600
_TRG = _V // _TRK


def _softmax_body(arch_ref, beta_ref, o_ref):
    a = arch_ref[...]
    b = beta_ref[0, 0]
    row = lax.broadcasted_iota(jnp.int32, (32, 1), 0)
    a = jnp.where(row < _F, a / b, -1e30)
    e = jnp.exp(a - jnp.max(a))
    p = e / jnp.sum(e)
    o_ref[...] = jnp.broadcast_to(p, (32, _D))


_mesh = plsc.VectorSubcoreMesh(
    core_axis_name="c", subcore_axis_name="s", num_cores=_NC, num_subcores=_NS
)

_PT = 7812
_G4L = 512
_NG4 = _PT // 4
_PER4 = _NG4 // _NW


def _dlane16():
    return plsc.cumsum(jnp.zeros((_D,), jnp.int32) + 1) - 1


@functools.partial(
    pl.kernel,
    out_type=jax.ShapeDtypeStruct((_V * _D // 128, 128), jnp.float32),
    mesh=_mesh,
    compiler_params=pltpu.CompilerParams(
        use_tc_tiling_on_sc=True, needs_layout_passes=False),
    scratch_types=[
        pltpu.VMEM((2, _D, _G4L), jnp.float32),
        pltpu.VMEM((2, _G4L * _D // 128, 129), jnp.float32),
        pltpu.VMEM((8, 128), jnp.float32),
        pltpu.SemaphoreType.DMA,
        pltpu.SemaphoreType.DMA,
        pltpu.SemaphoreType.DMA,
        pltpu.SemaphoreType.DMA,
    ],
)
def _sc_format_table(embt_hbm, tail_hbm, out_hbm,
                     slab, stage, tvm, si0, si1, so0, so1):
    wid = lax.axis_index("s") * _NC + lax.axis_index("c")
    base = wid * _PER4
    sems_i = (si0, si1)
    sems_o = (so0, so1)
    dlane = _dlane16()

    def in_cp(g, b):
        return pltpu.async_copy(
            embt_hbm.at[:, pl.ds(g * _G4L, _G4L)], slab.at[b], sems_i[b])

    def out_cp(g, b):
        return pltpu.async_copy(
            stage.at[b, :, pl.ds(0, 128)], out_hbm.at[pl.ds(g * 64, 64)],
            sems_o[b])

    lhi = jnp.where(dlane >= 8, 1, 0)
    colbase = (dlane - lhi * 8) * _D

    def compute(g, b):
        bsplat = jnp.zeros((_D,), jnp.int32) + b
        for d in range(_D):
            colv = colbase + d

            @plsc.parallel_loop(0, 32, unroll=8)
            def _mg(mg, b=b, d=d, colv=colv, bsplat=bsplat):
                v = slab[b, d, pl.ds(mg * _D, _D)]
                plsc.store_scatter(stage, [bsplat, lhi + 2 * mg, colv], v)

    in_cp(base + 0, 0)
    in_cp(base + 1, 1)

    def pair(p, carry):
        for b in (0, 1):
            g = base + p * 2 + b
            pltpu.make_async_copy(
                embt_hbm.at[:, pl.ds(g * _G4L, _G4L)], slab.at[b],
                sems_i[b]).wait()

            @pl.when(p >= 1)
            def _wait_out(g=g, b=b):
                pltpu.make_async_copy(
                    stage.at[b, :, pl.ds(0, 128)],
                    out_hbm.at[pl.ds((g - 2) * 64, 64)], sems_o[b]).wait()

            compute(g, b)
            out_cp(g, b)

            @pl.when(p * 2 + b <= _PER4 - 3)
            def _fire_next(g=g, b=b):
                in_cp(g + 2, b)
        return carry

    lax.fori_loop(0, _PER4 // 2, pair, 0)
    g_last = base + _PER4 - 1
    pltpu.make_async_copy(
        embt_hbm.at[:, pl.ds(g_last * _G4L, _G4L)], slab.at[0], sems_i[0]).wait()
    pltpu.make_async_copy(
        stage.at[0, :, pl.ds(0, 128)],
        out_hbm.at[pl.ds((g_last - 2) * 64, 64)], sems_o[0]).wait()
    compute(g_last, 0)
    out_cp(g_last, 0)
    pltpu.make_async_copy(
        stage.at[1, :, pl.ds(0, 128)],
        out_hbm.at[pl.ds((g_last - 1) * 64, 64)], sems_o[1]).wait()
    pltpu.make_async_copy(
        stage.at[0, :, pl.ds(0, 128)],
        out_hbm.at[pl.ds(g_last * 64, 64)], sems_o[0]).wait()

    @pl.when(wid == 0)
    def _leftover():
        g = _NG4 - 1
        pltpu.async_copy(
            embt_hbm.at[:, pl.ds(g * _G4L, _G4L)], slab.at[0], sems_i[0]).wait()
        compute(g, 0)
        pltpu.async_copy(
            stage.at[0, :, pl.ds(0, 128)],
            out_hbm.at[pl.ds(g * 64, 64)], sems_o[0]).wait()

    @pl.when(wid == _NW - 1)
    def _tail():
        pltpu.async_copy(tail_hbm, tvm, sems_i[1]).wait()
        pltpu.async_copy(
            tvm, out_hbm.at[pl.ds(_PT * _D, 8)], sems_o[1]).wait()


@functools.partial(
    pl.kernel,
    out_type=jax.ShapeDtypeStruct((_ROWS * _D // 1024, 8, 128), jnp.float32),
    mesh=_mesh,
    compiler_params=pltpu.CompilerParams(
        use_tc_tiling_on_sc=False, needs_layout_passes=False),
    scratch_types=[
        pltpu.VMEM((2 * _BC,), jnp.int32),
        pltpu.VMEM((2 * _BC, _D), jnp.float32),
        pltpu.VMEM((2, _NGC, _D, 129), jnp.float32),
        pltpu.VMEM((32, _D), jnp.float32),
        pltpu.SemaphoreType.DMA,
        pltpu.SemaphoreType.DMA,
        pltpu.SemaphoreType.DMA,
        pltpu.SemaphoreType.DMA,
    ],
)
def _sc_gather_scale(x_hbm, emb_hbm, ptab_hbm, out_hbm,
                     idx_v, rows_v, tbuf, ptile_v, sg0, sg1, so0, so1):
    wid = lax.axis_index("s") * _NC + lax.axis_index("c")
    b0 = wid * _BC
    sems_g = (sg0, sg1)
    sems_o = (so0, so1)

    pltpu.sync_copy(ptab_hbm, ptile_v)
    dlane = plsc.cumsum(jnp.zeros((_D,), jnp.int32) + 1) - 1

    def fire(f, s):
        pltpu.sync_copy(
            x_hbm.at[pl.ds(f * _B + b0, _BC)],
            idx_v.at[pl.ds(s * _BC, _BC)],
        )
        return [
            pltpu.async_copy(
                emb_hbm.at[idx_v.at[pl.ds(s * _BC + j * _GL, _GL)]],
                rows_v.at[pl.ds(s * _BC + j * _GL, _GL)],
                sems_g[s],
            )
            for j in range(_NGC)
        ]

    pend_g = fire(0, 0)
    pend_o = [None, None]
    for f in range(_F):
        s = f % 2
        nxt = fire(f + 1, 1 - s) if f + 1 < _F else None
        for c in pend_g:
            c.wait()
        if pend_o[s] is not None:
            for c in pend_o[s]:
                c.wait()
            pend_o[s] = None
        pv = ptile_v[f, :]
        zd = jnp.zeros((_D,), jnp.int32)
        zs = zd + s

        def _bt(bt, carry, s=s, pv=pv, zs=zs):
            rbase = s * _BC + bt * _GL
            zbt = zd + bt

            @plsc.parallel_loop(0, _GL, unroll=8)
            def _row(bl, rbase=rbase, zbt=zbt, pv=pv, zs=zs):
                v = rows_v[rbase + bl, :] * pv
                plsc.store_scatter(tbuf, [zs, zbt, dlane, zd + bl], v)

            return carry

        lax.fori_loop(0, _NGC, _bt, 0)
        ob = f * (_FSTRIDE // 1024) + wid * _NGC
        pend_o[s] = [
            pltpu.async_copy(
                tbuf.at[s, :, pl.ds(dt * 8, 8), pl.ds(0, 128)],
                out_hbm.at[pl.ds(ob + dt * (_DTSTRIDE // 1024), _NGC)],
                sems_o[s],
            )
            for dt in range(2)
        ]
        if nxt is not None:
            pend_g = nxt
    for po in pend_o:
        if po is not None:
            for c in po:
                c.wait()


def kernel(x, arch, embedding, beta):
    xlin = jnp.transpose(x.astype(jnp.int32)).reshape(_ROWS)
    arch_col = jnp.pad(arch.astype(jnp.float32), (0, 32 - _F)).reshape(32, 1)
    beta_arr = jnp.full((1, 1), beta, jnp.float32)
    ptab = pl.pallas_call(
        _softmax_body,
        out_shape=jax.ShapeDtypeStruct((32, _D), jnp.float32),
    )(arch_col, beta_arr)
    embT = jnp.transpose(embedding)
    tail = embedding[_PT * 128:, :].reshape(8, 128)
    emb2d = _sc_format_table(embT, tail)
    emb_rm = emb2d.reshape(_V, _D)
    out2d = _sc_gather_scale(xlin, emb_rm, ptab)
    out5 = out2d.reshape(_F, 2, _B // 128, 8, 128)
    return jnp.transpose(out5, (2, 4, 0, 1, 3)).reshape(_B, _F, _D)


# speedup vs baseline: 1.0131x; 1.0131x over previous
import functools

import jax
import jax.numpy as jnp
from jax import lax
from jax.experimental import pallas as pl
from jax.experimental.pallas import tpu as pltpu
from jax.experimental.pallas import tpu_sc as plsc

_B = 16384
_F = 26
_D = 16
_V = 1_000_000
_ROWS = _B * _F
_NC, _NS = 2, 16
_NW = _NC * _NS
_BC = _B // _NW
_GL = 128
_NGC = _BC // _GL
_TB = 2 * 8 * _BC
_FSTRIDE = _D * _B
_DTSTRIDE = 8 * _B

_TRK = 1600
_TRG = _V // _TRK


def _softmax_body(arch_ref, beta_ref, o_ref):
    a = arch_ref[...]
    b = beta_ref[0, 0]
    row = lax.broadcasted_iota(jnp.int32, (32, 1), 0)
    a = jnp.where(row < _F, a / b, -1e30)
    e = jnp.exp(a - jnp.max(a))
    p = e / jnp.sum(e)
    o_ref[...] = jnp.broadcast_to(p, (32, _D))


_mesh = plsc.VectorSubcoreMesh(
    core_axis_name="c", subcore_axis_name="s", num_cores=_NC, num_subcores=_NS
)

_PT = 7812
_G4L = 512
_NG4 = _PT // 4
_PER4 = _NG4 // _NW


def _dlane16():
    return plsc.cumsum(jnp.zeros((_D,), jnp.int32) + 1) - 1


@functools.partial(
    pl.kernel,
    out_type=jax.ShapeDtypeStruct((_V * _D // 128, 128), jnp.float32),
    mesh=_mesh,
    compiler_params=pltpu.CompilerParams(
        use_tc_tiling_on_sc=True, needs_layout_passes=False),
    scratch_types=[
        pltpu.VMEM((2, _D, _G4L), jnp.float32),
        pltpu.VMEM((2, _G4L * _D // 128, 128), jnp.float32),
        pltpu.VMEM((8, 128), jnp.float32),
        pltpu.SemaphoreType.DMA,
        pltpu.SemaphoreType.DMA,
        pltpu.SemaphoreType.DMA,
        pltpu.SemaphoreType.DMA,
    ],
)
def _sc_format_table(embt_hbm, tail_hbm, out_hbm,
                     slab, stage, tvm, si0, si1, so0, so1):
    wid = lax.axis_index("s") * _NC + lax.axis_index("c")
    base = wid * _PER4
    sems_i = (si0, si1)
    sems_o = (so0, so1)
    dlane = _dlane16()

    def in_cp(g, b):
        return pltpu.async_copy(
            embt_hbm.at[:, pl.ds(g * _G4L, _G4L)], slab.at[b], sems_i[b])

    def out_cp(g, b):
        return pltpu.async_copy(
            stage.at[b], out_hbm.at[pl.ds(g * 64, 64)],
            sems_o[b])

    lhi = jnp.where(dlane >= 8, 1, 0)
    colbase = (dlane - lhi * 8) * _D

    def compute(g, b):
        bsplat = jnp.zeros((_D,), jnp.int32) + b
        for d in range(_D):
            colv = colbase + d

            @plsc.parallel_loop(0, 32, unroll=8)
            def _mg(mg, b=b, d=d, colv=colv, bsplat=bsplat):
                v = slab[b, d, pl.ds(mg * _D, _D)]
                plsc.store_scatter(stage, [bsplat, lhi + 2 * mg, colv], v)

    in_cp(base + 0, 0)
    in_cp(base + 1, 1)

    def pair(p, carry):
        for b in (0, 1):
            g = base + p * 2 + b
            pltpu.make_async_copy(
                embt_hbm.at[:, pl.ds(g * _G4L, _G4L)], slab.at[b],
                sems_i[b]).wait()

            @pl.when(p >= 1)
            def _wait_out(g=g, b=b):
                pltpu.make_async_copy(
                    stage.at[b],
                    out_hbm.at[pl.ds((g - 2) * 64, 64)], sems_o[b]).wait()

            compute(g, b)
            out_cp(g, b)

            @pl.when(p * 2 + b <= _PER4 - 3)
            def _fire_next(g=g, b=b):
                in_cp(g + 2, b)
        return carry

    lax.fori_loop(0, _PER4 // 2, pair, 0)
    g_last = base + _PER4 - 1
    pltpu.make_async_copy(
        embt_hbm.at[:, pl.ds(g_last * _G4L, _G4L)], slab.at[0], sems_i[0]).wait()
    pltpu.make_async_copy(
        stage.at[0], out_hbm.at[pl.ds((g_last - 2) * 64, 64)], sems_o[0]).wait()
    compute(g_last, 0)
    out_cp(g_last, 0)
    pltpu.make_async_copy(
        stage.at[1], out_hbm.at[pl.ds((g_last - 1) * 64, 64)], sems_o[1]).wait()
    pltpu.make_async_copy(
        stage.at[0], out_hbm.at[pl.ds(g_last * 64, 64)], sems_o[0]).wait()

    @pl.when(wid == 0)
    def _leftover():
        g = _NG4 - 1
        pltpu.async_copy(
            embt_hbm.at[:, pl.ds(g * _G4L, _G4L)], slab.at[0], sems_i[0]).wait()
        compute(g, 0)
        pltpu.async_copy(
            stage.at[0], out_hbm.at[pl.ds(g * 64, 64)], sems_o[0]).wait()

    @pl.when(wid == _NW - 1)
    def _tail():
        pltpu.async_copy(tail_hbm, tvm, sems_i[1]).wait()
        pltpu.async_copy(
            tvm, out_hbm.at[pl.ds(_PT * _D, 8)], sems_o[1]).wait()


@functools.partial(
    pl.kernel,
    out_type=jax.ShapeDtypeStruct((_ROWS * _D // 1024, 8, 128), jnp.float32),
    mesh=_mesh,
    compiler_params=pltpu.CompilerParams(
        use_tc_tiling_on_sc=False, needs_layout_passes=False),
    scratch_types=[
        pltpu.VMEM((2 * _BC,), jnp.int32),
        pltpu.VMEM((2 * _BC, _D), jnp.float32),
        pltpu.VMEM((2, _NGC, _D, 129), jnp.float32),
        pltpu.VMEM((32, _D), jnp.float32),
        pltpu.SemaphoreType.DMA,
        pltpu.SemaphoreType.DMA,
        pltpu.SemaphoreType.DMA,
        pltpu.SemaphoreType.DMA,
    ],
)
def _sc_gather_scale(x_hbm, emb_hbm, ptab_hbm, out_hbm,
                     idx_v, rows_v, tbuf, ptile_v, sg0, sg1, so0, so1):
    wid = lax.axis_index("s") * _NC + lax.axis_index("c")
    b0 = wid * _BC
    sems_g = (sg0, sg1)
    sems_o = (so0, so1)

    pltpu.sync_copy(ptab_hbm, ptile_v)
    dlane = plsc.cumsum(jnp.zeros((_D,), jnp.int32) + 1) - 1

    def fire(f, s):
        pltpu.sync_copy(
            x_hbm.at[pl.ds(f * _B + b0, _BC)],
            idx_v.at[pl.ds(s * _BC, _BC)],
        )
        return [
            pltpu.async_copy(
                emb_hbm.at[idx_v.at[pl.ds(s * _BC + j * _GL, _GL)]],
                rows_v.at[pl.ds(s * _BC + j * _GL, _GL)],
                sems_g[s],
            )
            for j in range(_NGC)
        ]

    pend_g = fire(0, 0)
    pend_o = [None, None]
    for f in range(_F):
        s = f % 2
        nxt = fire(f + 1, 1 - s) if f + 1 < _F else None
        for c in pend_g:
            c.wait()
        if pend_o[s] is not None:
            for c in pend_o[s]:
                c.wait()
            pend_o[s] = None
        pv = ptile_v[f, :]
        zd = jnp.zeros((_D,), jnp.int32)
        zs = zd + s

        def _bt(bt, carry, s=s, pv=pv, zs=zs):
            rbase = s * _BC + bt * _GL
            zbt = zd + bt

            @plsc.parallel_loop(0, _GL, unroll=8)
            def _row(bl, rbase=rbase, zbt=zbt, pv=pv, zs=zs):
                v = rows_v[rbase + bl, :] * pv
                plsc.store_scatter(tbuf, [zs, zbt, dlane, zd + bl], v)

            return carry

        lax.fori_loop(0, _NGC, _bt, 0)
        ob = f * (_FSTRIDE // 1024) + wid * _NGC
        pend_o[s] = [
            pltpu.async_copy(
                tbuf.at[s, :, pl.ds(dt * 8, 8), pl.ds(0, 128)],
                out_hbm.at[pl.ds(ob + dt * (_DTSTRIDE // 1024), _NGC)],
                sems_o[s],
            )
            for dt in range(2)
        ]
        if nxt is not None:
            pend_g = nxt
    for po in pend_o:
        if po is not None:
            for c in po:
                c.wait()


def kernel(x, arch, embedding, beta):
    xlin = jnp.transpose(x.astype(jnp.int32)).reshape(_ROWS)
    arch_col = jnp.pad(arch.astype(jnp.float32), (0, 32 - _F)).reshape(32, 1)
    beta_arr = jnp.full((1, 1), beta, jnp.float32)
    ptab = pl.pallas_call(
        _softmax_body,
        out_shape=jax.ShapeDtypeStruct((32, _D), jnp.float32),
    )(arch_col, beta_arr)
    embT = jnp.transpose(embedding)
    tail = embedding[_PT * 128:, :].reshape(8, 128)
    emb2d = _sc_format_table(embT, tail)
    emb_rm = emb2d.reshape(_V, _D)
    out2d = _sc_gather_scale(xlin, emb_rm, ptab)
    out5 = out2d.reshape(_F, 2, _B // 128, 8, 128)
    return jnp.transpose(out5, (2, 4, 0, 1, 3)).reshape(_B, _F, _D)
